# baseline (speedup 1.0000x reference)
import functools

import jax
import jax.numpy as jnp
from jax import lax
from jax.experimental import pallas as pl
from jax.experimental.pallas import tpu as pltpu
from jax.experimental.pallas import tpu_sc as plsc

N = 10000
E = 320000
D = 128
DE = 16

NC = 2
NS = 16
NW = NC * NS
EB = 128
NBLK = E // EB
BPW = (NBLK + NW - 1) // NW
EBA = 80
NBLKA = E // EBA
BPWA = NBLKA // NW
NPAD = 10240
NPT = NPAD // NS

@functools.cache
def _mesh():
    return plsc.VectorSubcoreMesh(core_axis_name="c", subcore_axis_name="s")


def _msg_agg(h, e, src, dst, zeros):
    @functools.partial(
        pl.kernel,
        out_type=jax.ShapeDtypeStruct((NC, NPAD, D), jnp.float32),
        mesh=_mesh(),
        scratch_types=[
            pltpu.VMEM((EBA,), jnp.int32),
            pltpu.VMEM((EBA,), jnp.int32),
            pltpu.VMEM((EBA, D), jnp.float32),
            pltpu.VMEM((EBA, D), jnp.float32),
            pltpu.VMEM((EBA,), jnp.int32),
            pltpu.VMEM((EBA,), jnp.int32),
            pltpu.VMEM((EBA, D), jnp.float32),
            pltpu.VMEM((EBA, D), jnp.float32),
            pltpu.VMEM((EBA,), jnp.int32),
            pltpu.VMEM((EBA,), jnp.int32),
            pltpu.VMEM_SHARED((NPAD, D), jnp.float32),
            pltpu.SemaphoreType.DMA,
            pltpu.SemaphoreType.DMA,
            pltpu.SemaphoreType.DMA,
            pltpu.SemaphoreType.DMA,
            pltpu.SemaphoreType.DMA,
            pltpu.SemaphoreType.DMA,
            pltpu.SemaphoreType.DMA,
            pltpu.SemaphoreType.DMA,
        ],
    )
    def k(h_hbm, e_hbm, src_hbm, dst_hbm, z_hbm, agg_hbm,
          src0, dst0, hs0, ev0, src1, dst1, hs1, ev1, dsc0, dsc1, acc_sh,
          gs0, es0, gs1, es1, ss0, ss1, is0, is1):
        src_v = (src0, src1)
        dst_v = (dst0, dst1)
        dst_s = (dsc0, dsc1)
        hs_v = (hs0, hs1)
        e_v = (ev0, ev1)
        gsem = (gs0, gs1)
        esem = (es0, es1)
        ssem = (ss0, ss1)
        isem = (is0, is1)
        cid = lax.axis_index("c")
        sid = lax.axis_index("s")
        wid = sid * NC + cid
        pltpu.sync_copy(z_hbm.at[pl.ds(sid * NPT, NPT)],
                        acc_sh.at[pl.ds(sid * NPT, NPT)])
        plsc.subcore_barrier()

        def base_of(i):
            return (wid + i * NW) * EBA

        def idx_load(i, t):
            pltpu.async_copy(src_hbm.at[pl.ds(base_of(i), EBA)], src_v[t],
                             isem[t])
            pltpu.async_copy(dst_hbm.at[pl.ds(base_of(i), EBA)], dst_v[t],
                             isem[t])

        def idx_wait(i, t):
            pltpu.make_async_copy(src_hbm.at[pl.ds(base_of(i), EBA)],
                                  src_v[t], isem[t]).wait()
            pltpu.make_async_copy(dst_hbm.at[pl.ds(base_of(i), EBA)],
                                  dst_v[t], isem[t]).wait()

        def data_issue(i, t):
            pltpu.async_copy(h_hbm.at[src_v[t]], hs_v[t], gsem[t])
            pltpu.async_copy(e_hbm.at[pl.ds(base_of(i), EBA)], e_v[t],
                             esem[t])

        def data_wait(i, t):
            pltpu.make_async_copy(h_hbm.at[src_v[t]], hs_v[t],
                                  gsem[t]).wait()
            pltpu.make_async_copy(e_hbm.at[pl.ds(base_of(i), EBA)], e_v[t],
                                  esem[t]).wait()

        def scat_wait(t):
            pltpu.make_async_copy(e_v[t], acc_sh.at[dst_s[t]],
                                  ssem[t]).wait()

        idx_load(0, 0)
        idx_load(1, 1)
        idx_wait(0, 0)
        data_issue(0, 0)

        @pl.loop(0, (BPWA + 1) // 2)
        def _(it):
            for b in range(2):
                i = it * 2 + b

                @pl.when(i < BPWA)
                def _():
                    data_wait(i, b)

                @pl.when(i + 1 < BPWA)
                def _():
                    @pl.when(i >= 1)
                    def _():
                        scat_wait(1 - b)

                    idx_wait(i + 1, 1 - b)
                    data_issue(i + 1, 1 - b)

                @pl.when(i < BPWA)
                def _():
                    for j in range(EBA // 16):
                        sj = pl.ds(j * 16, 16)
                        dst_s[b].at[sj][...] = dst_v[b].at[sj][...]

                @pl.when(i + 2 < BPWA)
                def _():
                    idx_load(i + 2, b)

                @pl.when(i < BPWA)
                def _():
                    @pl.loop(0, EBA)
                    def _(r):
                        for j in range(D // 16):
                            sl = (pl.ds(r, 1), pl.ds(j * 16, 16))
                            e_v[b].at[*sl][...] = jnp.maximum(
                                e_v[b].at[*sl][...] + hs_v[b].at[*sl][...],
                                0.0)

                    pltpu.async_copy(e_v[b], acc_sh.at[dst_s[b]], ssem[b],
                                     add=True)

        scat_wait(0)
        scat_wait(1)
        plsc.subcore_barrier()
        pltpu.sync_copy(acc_sh.at[pl.ds(sid * NPT, NPT)],
                        agg_hbm.at[cid, pl.ds(sid * NPT, NPT)])

    return k(h, e, src, dst, zeros)


def _gather_pair_sum(a, b, src, dst):
    @functools.partial(
        pl.kernel,
        out_type=jax.ShapeDtypeStruct((E, D), jnp.float32),
        mesh=_mesh(),
        scratch_types=[
            pltpu.VMEM((EB,), jnp.int32),
            pltpu.VMEM((EB,), jnp.int32),
            pltpu.VMEM((EB, D), jnp.float32),
            pltpu.VMEM((EB, D), jnp.float32),
            pltpu.VMEM((EB,), jnp.int32),
            pltpu.VMEM((EB,), jnp.int32),
            pltpu.VMEM((EB, D), jnp.float32),
            pltpu.VMEM((EB, D), jnp.float32),
            pltpu.SemaphoreType.DMA,
            pltpu.SemaphoreType.DMA,
            pltpu.SemaphoreType.DMA,
            pltpu.SemaphoreType.DMA,
            pltpu.SemaphoreType.DMA,
            pltpu.SemaphoreType.DMA,
            pltpu.SemaphoreType.DMA,
            pltpu.SemaphoreType.DMA,
        ],
    )
    def k(a_hbm, b_hbm, src_hbm, dst_hbm, o_hbm,
          src0, dst0, av0, bv0, src1, dst1, av1, bv1,
          sa0, sb0, sa1, sb1, so0, so1, si0, si1):
        src_v = (src0, src1)
        dst_v = (dst0, dst1)
        av = (av0, av1)
        bv = (bv0, bv1)
        sa = (sa0, sa1)
        sb = (sb0, sb1)
        so = (so0, so1)
        si = (si0, si1)
        cid = lax.axis_index("c")
        sid = lax.axis_index("s")
        wid = sid * NC + cid
        nvalid = (NBLK + NW - 1 - wid) // NW

        def base_of(i):
            return (wid + i * NW) * EB

        def idx_load(i, t):
            pltpu.async_copy(src_hbm.at[pl.ds(base_of(i), EB)], src_v[t],
                             si[t])
            pltpu.async_copy(dst_hbm.at[pl.ds(base_of(i), EB)], dst_v[t],
                             si[t])

        def idx_wait(i, t):
            pltpu.make_async_copy(src_hbm.at[pl.ds(base_of(i), EB)],
                                  src_v[t], si[t]).wait()
            pltpu.make_async_copy(dst_hbm.at[pl.ds(base_of(i), EB)],
                                  dst_v[t], si[t]).wait()

        def data_issue(i, t):
            pltpu.async_copy(a_hbm.at[src_v[t]], av[t], sa[t])
            pltpu.async_copy(b_hbm.at[dst_v[t]], bv[t], sb[t])

        def data_wait(i, t):
            pltpu.make_async_copy(a_hbm.at[src_v[t]], av[t], sa[t]).wait()
            pltpu.make_async_copy(b_hbm.at[dst_v[t]], bv[t], sb[t]).wait()

        def out_wait(t):
            pltpu.make_async_copy(av[t], o_hbm.at[pl.ds(0, EB)],
                                  so[t]).wait()

        idx_load(0, 0)

        @pl.when(nvalid > 1)
        def _():
            idx_load(1, 1)

        idx_wait(0, 0)
        data_issue(0, 0)

        @pl.loop(0, (BPW + 1) // 2)
        def _(it):
            for b in range(2):
                i = it * 2 + b

                @pl.when(i < nvalid)
                def _():
                    data_wait(i, b)

                @pl.when(i + 1 < nvalid)
                def _():
                    @pl.when(i >= 1)
                    def _():
                        out_wait(1 - b)

                    idx_wait(i + 1, 1 - b)
                    data_issue(i + 1, 1 - b)

                @pl.when(i + 2 < nvalid)
                def _():
                    idx_load(i + 2, b)

                @pl.when(i < nvalid)
                def _():
                    @pl.loop(0, EB)
                    def _(r):
                        for j in range(D // 16):
                            sl = (pl.ds(r, 1), pl.ds(j * 16, 16))
                            av[b].at[*sl][...] = (av[b].at[*sl][...]
                                                  + bv[b].at[*sl][...])

                    pltpu.async_copy(av[b], o_hbm.at[pl.ds(base_of(i), EB)],
                                     so[b])

        out_wait(0)
        out_wait(1)

    return k(a, b, src, dst)


def _encode(v, w, bias, rows, block):
    din = v.shape[1]

    def body(v_ref, w_ref, b_ref, o_ref):
        o_ref[...] = jnp.dot(v_ref[...], w_ref[...],
                             preferred_element_type=jnp.float32) + b_ref[...]

    return pl.pallas_call(
        body,
        grid=(rows // block,),
        in_specs=[pl.BlockSpec((block, din), lambda i: (i, 0)),
                  pl.BlockSpec((din, D), lambda i: (0, 0)),
                  pl.BlockSpec((1, D), lambda i: (0, 0))],
        out_specs=pl.BlockSpec((block, D), lambda i: (i, 0)),
        out_shape=jax.ShapeDtypeStruct((rows, D), jnp.float32),
    )(v, w, bias.reshape(1, D))


_BN = 2000


def _node_mlp(z, W1, b1, W2, b2, g, be):
    z = jnp.maximum(jnp.dot(z, W1, preferred_element_type=jnp.float32) + b1,
                    0.0)
    z = jnp.dot(z, W2, preferred_element_type=jnp.float32) + b2
    mu = jnp.mean(z, axis=-1, keepdims=True)
    var = jnp.mean((z - mu) * (z - mu), axis=-1, keepdims=True)
    zn = g * (z - mu) / jnp.sqrt(var + 1e-5) + be
    return jnp.maximum(zn, 0.0)


def _node_update0(h, agg, one_eps, W1, b1, W2, b2, g, be, We1a, We1b):
    def body(h_ref, a0_ref, a1_ref, s_ref, W1r, b1r, W2r, b2r, gr, ber,
             war, wbr, hn_ref, a_ref, b_ref):
        z = s_ref[0, 0] * h_ref[...] + a0_ref[0] + a1_ref[0]
        hn = _node_mlp(z, W1r[...], b1r[...], W2r[...], b2r[...],
                       gr[...], ber[...])
        hn_ref[...] = hn
        a_ref[...] = jnp.dot(hn, war[...], preferred_element_type=jnp.float32)
        b_ref[...] = jnp.dot(hn, wbr[...], preferred_element_type=jnp.float32)

    full = lambda i: (0, 0)
    return pl.pallas_call(
        body,
        grid=(N // _BN,),
        in_specs=[pl.BlockSpec((_BN, D), lambda i: (i, 0)),
                  pl.BlockSpec((1, _BN, D), lambda i: (0, i, 0)),
                  pl.BlockSpec((1, _BN, D), lambda i: (1, i, 0)),
                  pl.BlockSpec((1, 1), full),
                  pl.BlockSpec((D, D), full),
                  pl.BlockSpec((1, D), full),
                  pl.BlockSpec((D, D), full),
                  pl.BlockSpec((1, D), full),
                  pl.BlockSpec((1, D), full),
                  pl.BlockSpec((1, D), full),
                  pl.BlockSpec((D, D), full),
                  pl.BlockSpec((D, D), full)],
        out_specs=[pl.BlockSpec((_BN, D), lambda i: (i, 0)),
                   pl.BlockSpec((_BN, D), lambda i: (i, 0)),
                   pl.BlockSpec((_BN, D), lambda i: (i, 0))],
        out_shape=[jax.ShapeDtypeStruct((N, D), jnp.float32),
                   jax.ShapeDtypeStruct((N, D), jnp.float32),
                   jax.ShapeDtypeStruct((N, D), jnp.float32)],
    )(h, agg, agg, one_eps.reshape(1, 1), W1, b1.reshape(1, D), W2,
      b2.reshape(1, D), g.reshape(1, D), be.reshape(1, D), We1a, We1b)


def _node_update1(h, agg, one_eps, W1, b1, W2, b2, g, be, Wh, bh):
    def body(h_ref, a0_ref, a1_ref, s_ref, W1r, b1r, W2r, b2r, gr, ber,
             whr, bhr, o_ref):
        z = s_ref[0, 0] * h_ref[...] + a0_ref[0] + a1_ref[0]
        hn = _node_mlp(z, W1r[...], b1r[...], W2r[...], b2r[...],
                       gr[...], ber[...])
        o_ref[...] = jnp.dot(hn, whr[...],
                             preferred_element_type=jnp.float32) + bhr[...]

    full = lambda i: (0, 0)
    return pl.pallas_call(
        body,
        grid=(N // _BN,),
        in_specs=[pl.BlockSpec((_BN, D), lambda i: (i, 0)),
                  pl.BlockSpec((1, _BN, D), lambda i: (0, i, 0)),
                  pl.BlockSpec((1, _BN, D), lambda i: (1, i, 0)),
                  pl.BlockSpec((1, 1), full),
                  pl.BlockSpec((D, D), full),
                  pl.BlockSpec((1, D), full),
                  pl.BlockSpec((D, D), full),
                  pl.BlockSpec((1, D), full),
                  pl.BlockSpec((1, D), full),
                  pl.BlockSpec((1, D), full),
                  pl.BlockSpec((D, 1), full),
                  pl.BlockSpec((1, 1), full)],
        out_specs=pl.BlockSpec((_BN, 1), lambda i: (i, 0)),
        out_shape=jax.ShapeDtypeStruct((N, 1), jnp.float32),
    )(h, agg, agg, one_eps.reshape(1, 1), W1, b1.reshape(1, D), W2,
      b2.reshape(1, D), g.reshape(1, D), be.reshape(1, D), Wh,
      bh.reshape(1, 1))


_BE = 2560


def _edge_update(e, gab, We1e, bE1, We2, bE2):
    def body(e_ref, g_ref, w1r, b1r, w2r, b2r, o_ref):
        t = jnp.dot(e_ref[...], w1r[...], preferred_element_type=jnp.float32)
        t = jnp.maximum(t + g_ref[...] + b1r[...], 0.0)
        o_ref[...] = e_ref[...] + (
            jnp.dot(t, w2r[...], preferred_element_type=jnp.float32)
            + b2r[...]) * 0.5

    full = lambda i: (0, 0)
    return pl.pallas_call(
        body,
        grid=(E // _BE,),
        in_specs=[pl.BlockSpec((_BE, D), lambda i: (i, 0)),
                  pl.BlockSpec((_BE, D), lambda i: (i, 0)),
                  pl.BlockSpec((D, D), full),
                  pl.BlockSpec((1, D), full),
                  pl.BlockSpec((D, D), full),
                  pl.BlockSpec((1, D), full)],
        out_specs=pl.BlockSpec((_BE, D), lambda i: (i, 0)),
        out_shape=jax.ShapeDtypeStruct((E, D), jnp.float32),
    )(e, gab, We1e, bE1.reshape(1, D), We2, bE2.reshape(1, D))


def kernel(x, edge_index, edge_attr, Wne, bne, Wee, bee,
           eps0, W1_0, b1_0, W2_0, b2_0, g_0, be_0, We1_0, be1_0, We2_0,
           be2_0, eps1, W1_1, b1_1, W2_1, b2_1, g_1, be_1, We1_1, be1_1,
           We2_1, be2_1, Wh, bh):
    src = edge_index[0]
    dst = edge_index[1]
    zeros = jnp.zeros((NPAD, D), jnp.float32)

    h = _encode(x, Wne, bne, N, _BN)
    e = _encode(edge_attr, Wee, bee, E, _BE)

    agg = _msg_agg(h, e, src, dst, zeros)
    h, A, B = _node_update0(h, agg, 1.0 + eps0, W1_0, b1_0, W2_0, b2_0,
                            g_0, be_0, We1_0[:D], We1_0[D:2 * D])
    gab = _gather_pair_sum(A, B, src, dst)
    e = _edge_update(e, gab, We1_0[2 * D:], be1_0, We2_0, be2_0)

    agg = _msg_agg(h, e, src, dst, zeros)
    return _node_update1(h, agg, 1.0 + eps1, W1_1, b1_1, W2_1, b2_1,
                         g_1, be_1, Wh, bh)


# speedup vs baseline: 1.0057x; 1.0057x over previous
import functools

import jax
import jax.numpy as jnp
from jax import lax
from jax.experimental import pallas as pl
from jax.experimental.pallas import tpu as pltpu
from jax.experimental.pallas import tpu_sc as plsc

N = 10000
E = 320000
D = 128
DE = 16

NC = 2
NS = 16
NW = NC * NS
EB = 200
NBLK = E // EB
BPW = (NBLK + NW - 1) // NW
EBA = 80
NBLKA = E // EBA
BPWA = NBLKA // NW
NPAD = 10240
NPT = NPAD // NS

@functools.cache
def _mesh():
    return plsc.VectorSubcoreMesh(core_axis_name="c", subcore_axis_name="s")


def _msg_agg(h, e, src, dst, zeros):
    @functools.partial(
        pl.kernel,
        out_type=jax.ShapeDtypeStruct((NC, NPAD, D), jnp.float32),
        mesh=_mesh(),
        scratch_types=[
            pltpu.VMEM((EBA,), jnp.int32),
            pltpu.VMEM((EBA,), jnp.int32),
            pltpu.VMEM((EBA, D), jnp.float32),
            pltpu.VMEM((EBA, D), jnp.float32),
            pltpu.VMEM((EBA,), jnp.int32),
            pltpu.VMEM((EBA,), jnp.int32),
            pltpu.VMEM((EBA, D), jnp.float32),
            pltpu.VMEM((EBA, D), jnp.float32),
            pltpu.VMEM((EBA,), jnp.int32),
            pltpu.VMEM((EBA,), jnp.int32),
            pltpu.VMEM_SHARED((NPAD, D), jnp.float32),
            pltpu.SemaphoreType.DMA,
            pltpu.SemaphoreType.DMA,
            pltpu.SemaphoreType.DMA,
            pltpu.SemaphoreType.DMA,
            pltpu.SemaphoreType.DMA,
            pltpu.SemaphoreType.DMA,
            pltpu.SemaphoreType.DMA,
            pltpu.SemaphoreType.DMA,
        ],
    )
    def k(h_hbm, e_hbm, src_hbm, dst_hbm, z_hbm, agg_hbm,
          src0, dst0, hs0, ev0, src1, dst1, hs1, ev1, dsc0, dsc1, acc_sh,
          gs0, es0, gs1, es1, ss0, ss1, is0, is1):
        src_v = (src0, src1)
        dst_v = (dst0, dst1)
        dst_s = (dsc0, dsc1)
        hs_v = (hs0, hs1)
        e_v = (ev0, ev1)
        gsem = (gs0, gs1)
        esem = (es0, es1)
        ssem = (ss0, ss1)
        isem = (is0, is1)
        cid = lax.axis_index("c")
        sid = lax.axis_index("s")
        wid = sid * NC + cid
        pltpu.sync_copy(z_hbm.at[pl.ds(sid * NPT, NPT)],
                        acc_sh.at[pl.ds(sid * NPT, NPT)])
        plsc.subcore_barrier()

        def base_of(i):
            return (wid + i * NW) * EBA

        def idx_load(i, t):
            pltpu.async_copy(src_hbm.at[pl.ds(base_of(i), EBA)], src_v[t],
                             isem[t])
            pltpu.async_copy(dst_hbm.at[pl.ds(base_of(i), EBA)], dst_v[t],
                             isem[t])

        def idx_wait(i, t):
            pltpu.make_async_copy(src_hbm.at[pl.ds(base_of(i), EBA)],
                                  src_v[t], isem[t]).wait()
            pltpu.make_async_copy(dst_hbm.at[pl.ds(base_of(i), EBA)],
                                  dst_v[t], isem[t]).wait()

        def data_issue(i, t):
            pltpu.async_copy(h_hbm.at[src_v[t]], hs_v[t], gsem[t])
            pltpu.async_copy(e_hbm.at[pl.ds(base_of(i), EBA)], e_v[t],
                             esem[t])

        def data_wait(i, t):
            pltpu.make_async_copy(h_hbm.at[src_v[t]], hs_v[t],
                                  gsem[t]).wait()
            pltpu.make_async_copy(e_hbm.at[pl.ds(base_of(i), EBA)], e_v[t],
                                  esem[t]).wait()

        def scat_wait(t):
            pltpu.make_async_copy(e_v[t], acc_sh.at[dst_s[t]],
                                  ssem[t]).wait()

        idx_load(0, 0)
        idx_load(1, 1)
        idx_wait(0, 0)
        data_issue(0, 0)

        @pl.loop(0, (BPWA + 1) // 2)
        def _(it):
            for b in range(2):
                i = it * 2 + b

                @pl.when(i < BPWA)
                def _():
                    data_wait(i, b)

                @pl.when(i + 1 < BPWA)
                def _():
                    @pl.when(i >= 1)
                    def _():
                        scat_wait(1 - b)

                    idx_wait(i + 1, 1 - b)
                    data_issue(i + 1, 1 - b)

                @pl.when(i < BPWA)
                def _():
                    for j in range(EBA // 16):
                        sj = pl.ds(j * 16, 16)
                        dst_s[b].at[sj][...] = dst_v[b].at[sj][...]

                @pl.when(i + 2 < BPWA)
                def _():
                    idx_load(i + 2, b)

                @pl.when(i < BPWA)
                def _():
                    @pl.loop(0, EBA)
                    def _(r):
                        for j in range(D // 16):
                            sl = (pl.ds(r, 1), pl.ds(j * 16, 16))
                            e_v[b].at[*sl][...] = jnp.maximum(
                                e_v[b].at[*sl][...] + hs_v[b].at[*sl][...],
                                0.0)

                    pltpu.async_copy(e_v[b], acc_sh.at[dst_s[b]], ssem[b],
                                     add=True)

        scat_wait(0)
        scat_wait(1)
        plsc.subcore_barrier()
        pltpu.sync_copy(acc_sh.at[pl.ds(sid * NPT, NPT)],
                        agg_hbm.at[cid, pl.ds(sid * NPT, NPT)])

    return k(h, e, src, dst, zeros)


def _gather_pair_sum(a, b, src, dst):
    @functools.partial(
        pl.kernel,
        out_type=jax.ShapeDtypeStruct((E, D), jnp.float32),
        mesh=_mesh(),
        scratch_types=[
            pltpu.VMEM((EB,), jnp.int32),
            pltpu.VMEM((EB,), jnp.int32),
            pltpu.VMEM((EB, D), jnp.float32),
            pltpu.VMEM((EB, D), jnp.float32),
            pltpu.VMEM((EB,), jnp.int32),
            pltpu.VMEM((EB,), jnp.int32),
            pltpu.VMEM((EB, D), jnp.float32),
            pltpu.VMEM((EB, D), jnp.float32),
            pltpu.SemaphoreType.DMA,
            pltpu.SemaphoreType.DMA,
            pltpu.SemaphoreType.DMA,
            pltpu.SemaphoreType.DMA,
            pltpu.SemaphoreType.DMA,
            pltpu.SemaphoreType.DMA,
            pltpu.SemaphoreType.DMA,
            pltpu.SemaphoreType.DMA,
        ],
    )
    def k(a_hbm, b_hbm, src_hbm, dst_hbm, o_hbm,
          src0, dst0, av0, bv0, src1, dst1, av1, bv1,
          sa0, sb0, sa1, sb1, so0, so1, si0, si1):
        src_v = (src0, src1)
        dst_v = (dst0, dst1)
        av = (av0, av1)
        bv = (bv0, bv1)
        sa = (sa0, sa1)
        sb = (sb0, sb1)
        so = (so0, so1)
        si = (si0, si1)
        cid = lax.axis_index("c")
        sid = lax.axis_index("s")
        wid = sid * NC + cid
        nvalid = (NBLK + NW - 1 - wid) // NW

        def base_of(i):
            return (wid + i * NW) * EB

        def idx_load(i, t):
            pltpu.async_copy(src_hbm.at[pl.ds(base_of(i), EB)], src_v[t],
                             si[t])
            pltpu.async_copy(dst_hbm.at[pl.ds(base_of(i), EB)], dst_v[t],
                             si[t])

        def idx_wait(i, t):
            pltpu.make_async_copy(src_hbm.at[pl.ds(base_of(i), EB)],
                                  src_v[t], si[t]).wait()
            pltpu.make_async_copy(dst_hbm.at[pl.ds(base_of(i), EB)],
                                  dst_v[t], si[t]).wait()

        def data_issue(i, t):
            pltpu.async_copy(a_hbm.at[src_v[t]], av[t], sa[t])
            pltpu.async_copy(b_hbm.at[dst_v[t]], bv[t], sb[t])

        def data_wait(i, t):
            pltpu.make_async_copy(a_hbm.at[src_v[t]], av[t], sa[t]).wait()
            pltpu.make_async_copy(b_hbm.at[dst_v[t]], bv[t], sb[t]).wait()

        def out_wait(t):
            pltpu.make_async_copy(av[t], o_hbm.at[pl.ds(0, EB)],
                                  so[t]).wait()

        idx_load(0, 0)

        @pl.when(nvalid > 1)
        def _():
            idx_load(1, 1)

        idx_wait(0, 0)
        data_issue(0, 0)

        @pl.loop(0, (BPW + 1) // 2)
        def _(it):
            for b in range(2):
                i = it * 2 + b

                @pl.when(i < nvalid)
                def _():
                    data_wait(i, b)

                @pl.when(i + 1 < nvalid)
                def _():
                    @pl.when(i >= 1)
                    def _():
                        out_wait(1 - b)

                    idx_wait(i + 1, 1 - b)
                    data_issue(i + 1, 1 - b)

                @pl.when(i + 2 < nvalid)
                def _():
                    idx_load(i + 2, b)

                @pl.when(i < nvalid)
                def _():
                    @pl.loop(0, EB)
                    def _(r):
                        for j in range(D // 16):
                            sl = (pl.ds(r, 1), pl.ds(j * 16, 16))
                            av[b].at[*sl][...] = (av[b].at[*sl][...]
                                                  + bv[b].at[*sl][...])

                    pltpu.async_copy(av[b], o_hbm.at[pl.ds(base_of(i), EB)],
                                     so[b])

        out_wait(0)
        out_wait(1)

    return k(a, b, src, dst)


def _encode(v, w, bias, rows, block):
    din = v.shape[1]

    def body(v_ref, w_ref, b_ref, o_ref):
        o_ref[...] = jnp.dot(v_ref[...], w_ref[...],
                             preferred_element_type=jnp.float32) + b_ref[...]

    return pl.pallas_call(
        body,
        grid=(rows // block,),
        in_specs=[pl.BlockSpec((block, din), lambda i: (i, 0)),
                  pl.BlockSpec((din, D), lambda i: (0, 0)),
                  pl.BlockSpec((1, D), lambda i: (0, 0))],
        out_specs=pl.BlockSpec((block, D), lambda i: (i, 0)),
        out_shape=jax.ShapeDtypeStruct((rows, D), jnp.float32),
    )(v, w, bias.reshape(1, D))


_BN = 2000


def _node_mlp(z, W1, b1, W2, b2, g, be):
    z = jnp.maximum(jnp.dot(z, W1, preferred_element_type=jnp.float32) + b1,
                    0.0)
    z = jnp.dot(z, W2, preferred_element_type=jnp.float32) + b2
    mu = jnp.mean(z, axis=-1, keepdims=True)
    var = jnp.mean((z - mu) * (z - mu), axis=-1, keepdims=True)
    zn = g * (z - mu) / jnp.sqrt(var + 1e-5) + be
    return jnp.maximum(zn, 0.0)


def _node_update0(h, agg, one_eps, W1, b1, W2, b2, g, be, We1a, We1b):
    def body(h_ref, a0_ref, a1_ref, s_ref, W1r, b1r, W2r, b2r, gr, ber,
             war, wbr, hn_ref, a_ref, b_ref):
        z = s_ref[0, 0] * h_ref[...] + a0_ref[0] + a1_ref[0]
        hn = _node_mlp(z, W1r[...], b1r[...], W2r[...], b2r[...],
                       gr[...], ber[...])
        hn_ref[...] = hn
        a_ref[...] = jnp.dot(hn, war[...], preferred_element_type=jnp.float32)
        b_ref[...] = jnp.dot(hn, wbr[...], preferred_element_type=jnp.float32)

    full = lambda i: (0, 0)
    return pl.pallas_call(
        body,
        grid=(N // _BN,),
        in_specs=[pl.BlockSpec((_BN, D), lambda i: (i, 0)),
                  pl.BlockSpec((1, _BN, D), lambda i: (0, i, 0)),
                  pl.BlockSpec((1, _BN, D), lambda i: (1, i, 0)),
                  pl.BlockSpec((1, 1), full),
                  pl.BlockSpec((D, D), full),
                  pl.BlockSpec((1, D), full),
                  pl.BlockSpec((D, D), full),
                  pl.BlockSpec((1, D), full),
                  pl.BlockSpec((1, D), full),
                  pl.BlockSpec((1, D), full),
                  pl.BlockSpec((D, D), full),
                  pl.BlockSpec((D, D), full)],
        out_specs=[pl.BlockSpec((_BN, D), lambda i: (i, 0)),
                   pl.BlockSpec((_BN, D), lambda i: (i, 0)),
                   pl.BlockSpec((_BN, D), lambda i: (i, 0))],
        out_shape=[jax.ShapeDtypeStruct((N, D), jnp.float32),
                   jax.ShapeDtypeStruct((N, D), jnp.float32),
                   jax.ShapeDtypeStruct((N, D), jnp.float32)],
    )(h, agg, agg, one_eps.reshape(1, 1), W1, b1.reshape(1, D), W2,
      b2.reshape(1, D), g.reshape(1, D), be.reshape(1, D), We1a, We1b)


def _node_update1(h, agg, one_eps, W1, b1, W2, b2, g, be, Wh, bh):
    def body(h_ref, a0_ref, a1_ref, s_ref, W1r, b1r, W2r, b2r, gr, ber,
             whr, bhr, o_ref):
        z = s_ref[0, 0] * h_ref[...] + a0_ref[0] + a1_ref[0]
        hn = _node_mlp(z, W1r[...], b1r[...], W2r[...], b2r[...],
                       gr[...], ber[...])
        o_ref[...] = jnp.dot(hn, whr[...],
                             preferred_element_type=jnp.float32) + bhr[...]

    full = lambda i: (0, 0)
    return pl.pallas_call(
        body,
        grid=(N // _BN,),
        in_specs=[pl.BlockSpec((_BN, D), lambda i: (i, 0)),
                  pl.BlockSpec((1, _BN, D), lambda i: (0, i, 0)),
                  pl.BlockSpec((1, _BN, D), lambda i: (1, i, 0)),
                  pl.BlockSpec((1, 1), full),
                  pl.BlockSpec((D, D), full),
                  pl.BlockSpec((1, D), full),
                  pl.BlockSpec((D, D), full),
                  pl.BlockSpec((1, D), full),
                  pl.BlockSpec((1, D), full),
                  pl.BlockSpec((1, D), full),
                  pl.BlockSpec((D, 1), full),
                  pl.BlockSpec((1, 1), full)],
        out_specs=pl.BlockSpec((_BN, 1), lambda i: (i, 0)),
        out_shape=jax.ShapeDtypeStruct((N, 1), jnp.float32),
    )(h, agg, agg, one_eps.reshape(1, 1), W1, b1.reshape(1, D), W2,
      b2.reshape(1, D), g.reshape(1, D), be.reshape(1, D), Wh,
      bh.reshape(1, 1))


_BE = 2560


def _edge_update(e, gab, We1e, bE1, We2, bE2):
    def body(e_ref, g_ref, w1r, b1r, w2r, b2r, o_ref):
        t = jnp.dot(e_ref[...], w1r[...], preferred_element_type=jnp.float32)
        t = jnp.maximum(t + g_ref[...] + b1r[...], 0.0)
        o_ref[...] = e_ref[...] + (
            jnp.dot(t, w2r[...], preferred_element_type=jnp.float32)
            + b2r[...]) * 0.5

    full = lambda i: (0, 0)
    return pl.pallas_call(
        body,
        grid=(E // _BE,),
        in_specs=[pl.BlockSpec((_BE, D), lambda i: (i, 0)),
                  pl.BlockSpec((_BE, D), lambda i: (i, 0)),
                  pl.BlockSpec((D, D), full),
                  pl.BlockSpec((1, D), full),
                  pl.BlockSpec((D, D), full),
                  pl.BlockSpec((1, D), full)],
        out_specs=pl.BlockSpec((_BE, D), lambda i: (i, 0)),
        out_shape=jax.ShapeDtypeStruct((E, D), jnp.float32),
    )(e, gab, We1e, bE1.reshape(1, D), We2, bE2.reshape(1, D))


def kernel(x, edge_index, edge_attr, Wne, bne, Wee, bee,
           eps0, W1_0, b1_0, W2_0, b2_0, g_0, be_0, We1_0, be1_0, We2_0,
           be2_0, eps1, W1_1, b1_1, W2_1, b2_1, g_1, be_1, We1_1, be1_1,
           We2_1, be2_1, Wh, bh):
    src = edge_index[0]
    dst = edge_index[1]
    zeros = jnp.zeros((NPAD, D), jnp.float32)

    h = _encode(x, Wne, bne, N, _BN)
    e = _encode(edge_attr, Wee, bee, E, _BE)

    agg = _msg_agg(h, e, src, dst, zeros)
    h, A, B = _node_update0(h, agg, 1.0 + eps0, W1_0, b1_0, W2_0, b2_0,
                            g_0, be_0, We1_0[:D], We1_0[D:2 * D])
    gab = _gather_pair_sum(A, B, src, dst)
    e = _edge_update(e, gab, We1_0[2 * D:], be1_0, We2_0, be2_0)

    agg = _msg_agg(h, e, src, dst, zeros)
    return _node_update1(h, agg, 1.0 + eps1, W1_1, b1_1, W2_1, b2_1,
                         g_1, be_1, Wh, bh)
